# baseline (speedup 1.0000x reference)
import jax
import jax.numpy as jnp
from jax import lax
from jax.experimental import pallas as pl
from jax.experimental.pallas import tpu as pltpu
from jax.experimental.pallas import tpu_sc as plsc

B = 4096
IN_FLAT = 352
HIDDEN = 1024
D = 256
K = 8192
COMMITMENT_COST = 0.25

BM = 1024
NB = B // BM
KB = 2048
NK = K // KB

_PREC = lax.Precision.HIGHEST
_BF = jnp.bfloat16


def _bdot(a, b, dims):
    return lax.dot_general(a.astype(_BF), b.astype(_BF), dims,
                           preferred_element_type=jnp.float32)


def _enc_argmin_body(x_ref, W1_ref, b1_ref, W2_ref, b2_ref, cb_ref,
                     idx_ref, loss_ref, cbm2_ref, cn_ref):
    i = pl.program_id(0)

    @pl.when(i == 0)
    def _():
        cbf = cb_ref[...]
        cbm2_ref[...] = (cbf * -2.0).astype(_BF)
        ones_row = jnp.ones((1, D), dtype=jnp.float32)
        cn_ref[...] = lax.dot_general(
            ones_row, cbf * cbf, (((1,), (1,)), ((), ())),
            precision=_PREC, preferred_element_type=jnp.float32)
        loss_ref[...] = jnp.zeros_like(loss_ref)

    h = jnp.maximum(
        _bdot(x_ref[...], W1_ref[...], (((1,), (0,)), ((), ()))) + b1_ref[...],
        0.0)
    e = _bdot(h, W2_ref[...], (((1,), (0,)), ((), ()))) + b2_ref[...]
    e_bf = e.astype(_BF)
    en = jnp.sum(e * e, axis=1, keepdims=True)
    iotaf = lax.broadcasted_iota(jnp.int32, (BM, KB), 1).astype(jnp.float32)

    best = jnp.full((BM, 1), jnp.inf, dtype=jnp.float32)
    bidx = jnp.zeros((BM, 1), dtype=jnp.float32)
    for j in range(NK):
        sc = lax.dot_general(e_bf, cbm2_ref[j * KB:(j + 1) * KB, :],
                             (((1,), (1,)), ((), ())),
                             preferred_element_type=jnp.float32)
        d = (en + sc) + cn_ref[:, j * KB:(j + 1) * KB]
        bm = jnp.min(d, axis=1, keepdims=True)
        la = jnp.min(jnp.where(d == bm, iotaf, float(KB)),
                     axis=1, keepdims=True)
        upd = bm < best
        bidx = jnp.where(upd, la + float(j * KB), bidx)
        best = jnp.where(upd, bm, best)

    idx_ref[...] = bidx.astype(jnp.int32)
    loss_ref[...] += jnp.sum(best)

    @pl.when(i == NB - 1)
    def _():
        loss_ref[...] *= COMMITMENT_COST / (B * D)


def _decoder_body(q_ref, W1_ref, b1_ref, W2_ref, b2_ref, out_ref):
    h = jnp.maximum(
        _bdot(q_ref[...], W1_ref[...], (((1,), (0,)), ((), ()))) + b1_ref[...],
        0.0)
    dec = _bdot(h, W2_ref[...], (((1,), (0,)), ((), ()))) + b2_ref[...]
    out_ref[...] = dec.reshape(BM, 4, 88)


_NC = 2
_NS = 16
_NW = _NC * _NS
_BPW = B // _NW


def _sc_gather_body(cb_hbm, idx_hbm, out_hbm, idx_v, rows_v, sem):
    wid = lax.axis_index("s") * _NC + lax.axis_index("c")
    base = wid * _BPW
    pltpu.sync_copy(idx_hbm.at[pl.ds(base, _BPW)], idx_v)
    pltpu.async_copy(cb_hbm.at[idx_v], rows_v, sem).wait()
    pltpu.sync_copy(rows_v, out_hbm.at[pl.ds(base, _BPW)])


def _sc_gather(codebook, idx):
    return pl.kernel(
        _sc_gather_body,
        out_type=jax.ShapeDtypeStruct((B, D), jnp.float32),
        mesh=plsc.VectorSubcoreMesh(core_axis_name="c", subcore_axis_name="s"),
        scratch_types=[
            pltpu.VMEM((_BPW,), jnp.int32),
            pltpu.VMEM((_BPW, D), jnp.float32),
            pltpu.SemaphoreType.DMA,
        ],
    )(codebook, idx)


def kernel(x, W_enc1, b_enc1, W_enc2, b_enc2, codebook,
           W_dec1, b_dec1, W_dec2, b_dec2):
    xf = x.reshape(B, IN_FLAT)

    idx2d, loss_sum = pl.pallas_call(
        _enc_argmin_body,
        grid=(NB,),
        in_specs=[
            pl.BlockSpec((BM, IN_FLAT), lambda i: (i, 0)),
            pl.BlockSpec((IN_FLAT, HIDDEN), lambda i: (0, 0)),
            pl.BlockSpec((1, HIDDEN), lambda i: (0, 0)),
            pl.BlockSpec((HIDDEN, D), lambda i: (0, 0)),
            pl.BlockSpec((1, D), lambda i: (0, 0)),
            pl.BlockSpec((K, D), lambda i: (0, 0)),
        ],
        out_specs=[
            pl.BlockSpec((BM, 1), lambda i: (i, 0)),
            pl.BlockSpec((1, 1), lambda i: (0, 0)),
        ],
        out_shape=[
            jax.ShapeDtypeStruct((B, 1), jnp.int32),
            jax.ShapeDtypeStruct((1, 1), jnp.float32),
        ],
        scratch_shapes=[
            pltpu.VMEM((K, D), _BF),
            pltpu.VMEM((1, K), jnp.float32),
        ],
    )(xf, W_enc1, b_enc1.reshape(1, HIDDEN), W_enc2, b_enc2.reshape(1, D),
      codebook)

    idx = idx2d.reshape(B)
    quantized = _sc_gather(codebook, idx)

    decoded = pl.pallas_call(
        _decoder_body,
        grid=(NB,),
        in_specs=[
            pl.BlockSpec((BM, D), lambda i: (i, 0)),
            pl.BlockSpec((D, HIDDEN), lambda i: (0, 0)),
            pl.BlockSpec((1, HIDDEN), lambda i: (0, 0)),
            pl.BlockSpec((HIDDEN, IN_FLAT), lambda i: (0, 0)),
            pl.BlockSpec((1, IN_FLAT), lambda i: (0, 0)),
        ],
        out_specs=pl.BlockSpec((BM, 4, 88), lambda i: (i, 0, 0)),
        out_shape=jax.ShapeDtypeStruct((B, 4, 88), jnp.float32),
    )(quantized, W_dec1, b_dec1.reshape(1, HIDDEN), W_dec2,
      b_dec2.reshape(1, IN_FLAT))

    return (decoded, loss_sum[0, 0])


# speedup vs baseline: 1.0282x; 1.0282x over previous
import jax
import jax.numpy as jnp
from jax import lax
from jax.experimental import pallas as pl
from jax.experimental.pallas import tpu as pltpu
from jax.experimental.pallas import tpu_sc as plsc

B = 4096
IN_FLAT = 352
HIDDEN = 1024
D = 256
K = 8192
COMMITMENT_COST = 0.25

BM = 2048
NB = B // BM
KB = 2048
NK = K // KB

_PREC = lax.Precision.HIGHEST
_BF = jnp.bfloat16


def _bdot(a, b, dims):
    return lax.dot_general(a.astype(_BF), b.astype(_BF), dims,
                           preferred_element_type=jnp.float32)


def _enc_argmin_body(x_ref, W1_ref, b1_ref, W2_ref, b2_ref, cb_ref,
                     idx_ref, loss_ref, cbm2_ref, cn_ref):
    i = pl.program_id(0)

    @pl.when(i == 0)
    def _():
        cbf = cb_ref[...]
        cbm2_ref[...] = (cbf * -2.0).astype(_BF)
        ones_row = jnp.ones((1, D), dtype=jnp.float32)
        cn_ref[...] = lax.dot_general(
            ones_row, cbf * cbf, (((1,), (1,)), ((), ())),
            precision=_PREC, preferred_element_type=jnp.float32)
        loss_ref[...] = jnp.zeros_like(loss_ref)

    h = jnp.maximum(
        _bdot(x_ref[...], W1_ref[...], (((1,), (0,)), ((), ()))) + b1_ref[...],
        0.0)
    e = _bdot(h, W2_ref[...], (((1,), (0,)), ((), ()))) + b2_ref[...]
    e_bf = e.astype(_BF)
    en = jnp.sum(e * e, axis=1, keepdims=True)
    iotaf = lax.broadcasted_iota(jnp.int32, (BM, KB), 1).astype(jnp.float32)

    best = jnp.full((BM, 1), jnp.inf, dtype=jnp.float32)
    bidx = jnp.zeros((BM, 1), dtype=jnp.float32)
    for j in range(NK):
        sc = lax.dot_general(e_bf, cbm2_ref[j * KB:(j + 1) * KB, :],
                             (((1,), (1,)), ((), ())),
                             preferred_element_type=jnp.float32)
        d = (en + sc) + cn_ref[:, j * KB:(j + 1) * KB]
        bm = jnp.min(d, axis=1, keepdims=True)
        la = jnp.min(jnp.where(d == bm, iotaf, float(KB)),
                     axis=1, keepdims=True)
        upd = bm < best
        bidx = jnp.where(upd, la + float(j * KB), bidx)
        best = jnp.where(upd, bm, best)

    idx_ref[...] = bidx.astype(jnp.int32)
    loss_ref[...] += jnp.sum(best)


def _decoder_body(q_ref, W1_ref, b1_ref, W2_ref, b2_ref, out_ref):
    h = jnp.maximum(
        _bdot(q_ref[...], W1_ref[...], (((1,), (0,)), ((), ()))) + b1_ref[...],
        0.0)
    out_ref[...] = (
        _bdot(h, W2_ref[...], (((1,), (0,)), ((), ()))) + b2_ref[...])


_NC = 2
_NS = 16
_NW = _NC * _NS
_BPW = B // _NW


def _sc_gather_body(cb_hbm, idx_hbm, out_hbm, idx_v, rows_v, sem):
    wid = lax.axis_index("s") * _NC + lax.axis_index("c")
    base = wid * _BPW
    pltpu.sync_copy(idx_hbm.at[pl.ds(base, _BPW)], idx_v)
    pltpu.async_copy(cb_hbm.at[idx_v], rows_v, sem).wait()
    pltpu.sync_copy(rows_v, out_hbm.at[pl.ds(base, _BPW)])


def _sc_gather(codebook, idx):
    return pl.kernel(
        _sc_gather_body,
        out_type=jax.ShapeDtypeStruct((B, D), jnp.float32),
        mesh=plsc.VectorSubcoreMesh(core_axis_name="c", subcore_axis_name="s"),
        scratch_types=[
            pltpu.VMEM((_BPW,), jnp.int32),
            pltpu.VMEM((_BPW, D), jnp.float32),
            pltpu.SemaphoreType.DMA,
        ],
    )(codebook, idx)


def kernel(x, W_enc1, b_enc1, W_enc2, b_enc2, codebook,
           W_dec1, b_dec1, W_dec2, b_dec2):
    xf = x.reshape(B, IN_FLAT)

    idx2d, loss_sum = pl.pallas_call(
        _enc_argmin_body,
        grid=(NB,),
        in_specs=[
            pl.BlockSpec((BM, IN_FLAT), lambda i: (i, 0)),
            pl.BlockSpec((IN_FLAT, HIDDEN), lambda i: (0, 0)),
            pl.BlockSpec((1, HIDDEN), lambda i: (0, 0)),
            pl.BlockSpec((HIDDEN, D), lambda i: (0, 0)),
            pl.BlockSpec((1, D), lambda i: (0, 0)),
            pl.BlockSpec((K, D), lambda i: (0, 0)),
        ],
        out_specs=[
            pl.BlockSpec((BM, 1), lambda i: (i, 0)),
            pl.BlockSpec((1, 1), lambda i: (0, 0)),
        ],
        out_shape=[
            jax.ShapeDtypeStruct((B, 1), jnp.int32),
            jax.ShapeDtypeStruct((1, 1), jnp.float32),
        ],
        scratch_shapes=[
            pltpu.VMEM((K, D), _BF),
            pltpu.VMEM((1, K), jnp.float32),
        ],
    )(xf, W_enc1, b_enc1.reshape(1, HIDDEN), W_enc2, b_enc2.reshape(1, D),
      codebook)

    idx = idx2d.reshape(B)
    quantized = _sc_gather(codebook, idx)

    decoded = pl.pallas_call(
        _decoder_body,
        grid=(NB,),
        in_specs=[
            pl.BlockSpec((BM, D), lambda i: (i, 0)),
            pl.BlockSpec((D, HIDDEN), lambda i: (0, 0)),
            pl.BlockSpec((1, HIDDEN), lambda i: (0, 0)),
            pl.BlockSpec((HIDDEN, IN_FLAT), lambda i: (0, 0)),
            pl.BlockSpec((1, IN_FLAT), lambda i: (0, 0)),
        ],
        out_specs=pl.BlockSpec((BM, IN_FLAT), lambda i: (i, 0)),
        out_shape=jax.ShapeDtypeStruct((B, IN_FLAT), jnp.float32),
    )(quantized, W_dec1, b_dec1.reshape(1, HIDDEN), W_dec2,
      b_dec2.reshape(1, IN_FLAT))

    vq_loss = (COMMITMENT_COST / (B * D)) * loss_sum[0, 0]
    return (decoded.reshape(B, 4, 88), vq_loss)


# speedup vs baseline: 1.0524x; 1.0235x over previous
import jax
import jax.numpy as jnp
from jax import lax
from jax.experimental import pallas as pl
from jax.experimental.pallas import tpu as pltpu
from jax.experimental.pallas import tpu_sc as plsc

B = 4096
IN_FLAT = 352
HIDDEN = 1024
D = 256
K = 8192
COMMITMENT_COST = 0.25

BM = 1024
NB = B // BM
KB = 1024
NK = K // KB

_PREC = lax.Precision.HIGHEST
_BF = jnp.bfloat16


def _bdot(a, b, dims):
    return lax.dot_general(a.astype(_BF), b.astype(_BF), dims,
                           preferred_element_type=jnp.float32)


def _enc_argmin_body(x_ref, W1_ref, b1_ref, W2_ref, b2_ref, cb_ref,
                     idx_ref, loss_ref, cbm2_ref, cn_ref):
    i = pl.program_id(0)

    @pl.when(i == 0)
    def _():
        cbf = cb_ref[...]
        cbm2_ref[...] = (cbf * -2.0).astype(_BF)
        ones_row = jnp.ones((1, D), dtype=jnp.float32)
        cn_ref[...] = lax.dot_general(
            ones_row, cbf * cbf, (((1,), (1,)), ((), ())),
            precision=_PREC, preferred_element_type=jnp.float32)
        loss_ref[...] = jnp.zeros_like(loss_ref)

    h = jnp.maximum(
        _bdot(x_ref[...], W1_ref[...], (((1,), (0,)), ((), ()))) + b1_ref[...],
        0.0)
    e = _bdot(h, W2_ref[...], (((1,), (0,)), ((), ()))) + b2_ref[...]
    e_bf = e.astype(_BF)
    en = jnp.sum(e * e, axis=1, keepdims=True)
    iotaf = lax.broadcasted_iota(jnp.int32, (BM, KB), 1).astype(jnp.float32)

    best = jnp.full((BM, 1), jnp.inf, dtype=jnp.float32)
    bidx = jnp.zeros((BM, 1), dtype=jnp.float32)
    for j in range(NK):
        sc = lax.dot_general(e_bf, cbm2_ref[j * KB:(j + 1) * KB, :],
                             (((1,), (1,)), ((), ())),
                             preferred_element_type=jnp.float32)
        d = (en + sc) + cn_ref[:, j * KB:(j + 1) * KB]
        bm = jnp.min(d, axis=1, keepdims=True)
        la = jnp.min(jnp.where(d == bm, iotaf, float(KB)),
                     axis=1, keepdims=True)
        upd = bm < best
        bidx = jnp.where(upd, la + float(j * KB), bidx)
        best = jnp.where(upd, bm, best)

    idx_ref[...] = bidx.astype(jnp.int32)
    loss_ref[...] += jnp.sum(best)


def _decoder_body(q_ref, W1_ref, b1_ref, W2_ref, b2_ref, out_ref):
    h = jnp.maximum(
        _bdot(q_ref[...], W1_ref[...], (((1,), (0,)), ((), ()))) + b1_ref[...],
        0.0)
    out_ref[...] = (
        _bdot(h, W2_ref[...], (((1,), (0,)), ((), ()))) + b2_ref[...])


_NC = 2
_NS = 16
_NW = _NC * _NS
_BPW = B // _NW


def _sc_gather_body(cb_hbm, idx_hbm, out_hbm, idx_v, rows_v, sem):
    wid = lax.axis_index("s") * _NC + lax.axis_index("c")
    base = wid * _BPW
    pltpu.sync_copy(idx_hbm.at[pl.ds(base, _BPW)], idx_v)
    pltpu.async_copy(cb_hbm.at[idx_v], rows_v, sem).wait()
    pltpu.sync_copy(rows_v, out_hbm.at[pl.ds(base, _BPW)])


def _sc_gather(codebook, idx):
    return pl.kernel(
        _sc_gather_body,
        out_type=jax.ShapeDtypeStruct((B, D), jnp.float32),
        mesh=plsc.VectorSubcoreMesh(core_axis_name="c", subcore_axis_name="s"),
        scratch_types=[
            pltpu.VMEM((_BPW,), jnp.int32),
            pltpu.VMEM((_BPW, D), jnp.float32),
            pltpu.SemaphoreType.DMA,
        ],
    )(codebook, idx)


def kernel(x, W_enc1, b_enc1, W_enc2, b_enc2, codebook,
           W_dec1, b_dec1, W_dec2, b_dec2):
    xf = x.reshape(B, IN_FLAT)

    idx2d, loss_sum = pl.pallas_call(
        _enc_argmin_body,
        grid=(NB,),
        in_specs=[
            pl.BlockSpec((BM, IN_FLAT), lambda i: (i, 0)),
            pl.BlockSpec((IN_FLAT, HIDDEN), lambda i: (0, 0)),
            pl.BlockSpec((1, HIDDEN), lambda i: (0, 0)),
            pl.BlockSpec((HIDDEN, D), lambda i: (0, 0)),
            pl.BlockSpec((1, D), lambda i: (0, 0)),
            pl.BlockSpec((K, D), lambda i: (0, 0)),
        ],
        out_specs=[
            pl.BlockSpec((BM, 1), lambda i: (i, 0)),
            pl.BlockSpec((1, 1), lambda i: (0, 0)),
        ],
        out_shape=[
            jax.ShapeDtypeStruct((B, 1), jnp.int32),
            jax.ShapeDtypeStruct((1, 1), jnp.float32),
        ],
        scratch_shapes=[
            pltpu.VMEM((K, D), _BF),
            pltpu.VMEM((1, K), jnp.float32),
        ],
    )(xf, W_enc1, b_enc1.reshape(1, HIDDEN), W_enc2, b_enc2.reshape(1, D),
      codebook)

    idx = idx2d.reshape(B)
    quantized = _sc_gather(codebook, idx)

    decoded = pl.pallas_call(
        _decoder_body,
        grid=(NB,),
        in_specs=[
            pl.BlockSpec((BM, D), lambda i: (i, 0)),
            pl.BlockSpec((D, HIDDEN), lambda i: (0, 0)),
            pl.BlockSpec((1, HIDDEN), lambda i: (0, 0)),
            pl.BlockSpec((HIDDEN, IN_FLAT), lambda i: (0, 0)),
            pl.BlockSpec((1, IN_FLAT), lambda i: (0, 0)),
        ],
        out_specs=pl.BlockSpec((BM, IN_FLAT), lambda i: (i, 0)),
        out_shape=jax.ShapeDtypeStruct((B, IN_FLAT), jnp.float32),
    )(quantized, W_dec1, b_dec1.reshape(1, HIDDEN), W_dec2,
      b_dec2.reshape(1, IN_FLAT))

    vq_loss = (COMMITMENT_COST / (B * D)) * loss_sum[0, 0]
    return (decoded.reshape(B, 4, 88), vq_loss)
